# Initial kernel scaffold; baseline (speedup 1.0000x reference)
import jax
import jax.numpy as jnp
from jax.experimental import pallas as pl


def kernel(entity_emb, weight, qTrans, kTrans, vTrans, edge_index, edge_type):
    raise NotImplementedError("write your pallas kernel here")


# speedup vs baseline: 6.4066x; 6.4066x over previous
import functools

import jax
import jax.numpy as jnp
from jax import lax
from jax.experimental import pallas as pl
from jax.experimental.pallas import tpu as pltpu
from jax.experimental.pallas import tpu_sc as plsc

N_NODES = 10000
N_EDGES = 320000
DIM = 128
HEADS = 4
HDIM = 32
NREL = 9

NC = 2
NS = 16
L = 16
NW = NC * NS

EPW = N_EDGES // NW
CHUNK = 80
NCHUNK = EPW // CHUNK
GROUPS = CHUNK // L
NDRAIN_SC = 10
ROWS_PT = N_NODES // NDRAIN_SC
BOUNCE = 40
DRAIN = 200

_MESH = dict(core_axis_name="c", subcore_axis_name="s")


def _zero_vmem_2d(ref, rows, cols):
    zero = jnp.zeros((L,), jnp.float32)

    def body(i, _):
        for k in range(cols // L):
            ref[i, pl.ds(k * L, L)] = zero
        return 0

    lax.fori_loop(0, rows, body, 0)


def _tc_q(emb, qT):
    BN, NB = 1000, 10

    def body(e_ref, q_ref, o_ref):
        o_ref[...] = jnp.dot(e_ref[...], q_ref[...],
                             preferred_element_type=jnp.float32)

    return pl.pallas_call(
        body,
        grid=(NB,),
        in_specs=[pl.BlockSpec((BN, DIM), lambda b: (b, 0)),
                  pl.BlockSpec((DIM, DIM), lambda b: (0, 0))],
        out_specs=pl.BlockSpec((BN, DIM), lambda b: (b, 0)),
        out_shape=jax.ShapeDtypeStruct((N_NODES, DIM), jnp.float32),
    )(emb, qT)


def _tc_tables(wpad, emb, kT, vT):
    BN, NB = 1000, 10

    def body(w_ref, e_ref, k_ref, v_ref, ko_ref, vo_ref):
        r = pl.program_id(0)
        x = e_ref[...] * w_ref[pl.ds(r, 1), :]
        ko_ref[...] = jnp.dot(x, k_ref[...],
                              preferred_element_type=jnp.float32)
        vo_ref[...] = jnp.dot(x, v_ref[...],
                              preferred_element_type=jnp.float32)

    return pl.pallas_call(
        body,
        grid=(NREL, NB),
        in_specs=[pl.BlockSpec((L, DIM), lambda r, b: (0, 0)),
                  pl.BlockSpec((BN, DIM), lambda r, b: (b, 0)),
                  pl.BlockSpec((DIM, DIM), lambda r, b: (0, 0)),
                  pl.BlockSpec((DIM, DIM), lambda r, b: (0, 0))],
        out_specs=[pl.BlockSpec((BN, DIM), lambda r, b: (r * NB + b, 0)),
                   pl.BlockSpec((BN, DIM), lambda r, b: (r * NB + b, 0))],
        out_shape=[jax.ShapeDtypeStruct((NREL * N_NODES, DIM), jnp.float32),
                   jax.ShapeDtypeStruct((NREL * N_NODES, DIM), jnp.float32)],
    )(wpad, emb, kT, vT)


def _tc_finalize_g(wpad, kgU, attN):
    BN, NB = 1000, 10

    def body(w_ref, k2_ref, a2_ref, g_ref):
        ku = k2_ref[0] + k2_ref[1]
        an = a2_ref[0] + a2_ref[1]
        r16 = lax.broadcasted_iota(jnp.int32, (L, DIM), 0)
        c128 = lax.broadcasted_iota(jnp.int32, (L, DIM), 1)
        sel = (c128 // HDIM == r16).astype(jnp.float32)
        div = jnp.dot(an, sel, preferred_element_type=jnp.float32) + 1e-8
        kg = ku / div
        kg2 = kg * kg
        w2 = w_ref[...] * w_ref[...]
        g_ref[...] = lax.dot_general(kg2, w2, (((1,), (1,)), ((), ())),
                                     preferred_element_type=jnp.float32)

    return pl.pallas_call(
        body,
        grid=(NB,),
        in_specs=[pl.BlockSpec((L, DIM), lambda b: (0, 0)),
                  pl.BlockSpec((NC, BN, DIM), lambda b: (0, b, 0)),
                  pl.BlockSpec((NC, BN, L), lambda b: (0, b, 0))],
        out_specs=pl.BlockSpec((BN, L), lambda b: (b, 0)),
        out_shape=jax.ShapeDtypeStruct((N_NODES, L), jnp.float32),
    )(wpad, kgU, attN)


def _tc_final(outU, sU):
    BN, NB = 1000, 10

    def body(u_ref, s_ref, o_ref):
        u = u_ref[0] + u_ref[1]
        sv = s_ref[0] + s_ref[1]
        sel0 = (lax.broadcasted_iota(jnp.int32, (L, DIM), 0) == 0
                ).astype(jnp.float32)
        s128 = jnp.dot(sv, sel0, preferred_element_type=jnp.float32)
        o_ref[...] = u / (s128 + 1e-16)

    return pl.pallas_call(
        body,
        grid=(NB,),
        in_specs=[pl.BlockSpec((NC, BN, DIM), lambda b: (0, b, 0)),
                  pl.BlockSpec((NC, BN, L), lambda b: (0, b, 0))],
        out_specs=pl.BlockSpec((BN, DIM), lambda b: (b, 0)),
        out_shape=jax.ShapeDtypeStruct((N_NODES, DIM), jnp.float32),
    )(outU, sU)


_SC_PARAMS = pltpu.CompilerParams(use_tc_tiling_on_sc=False)


@functools.partial(
    pl.kernel,
    out_type=(jax.ShapeDtypeStruct((NC, N_NODES, DIM), jnp.float32),
              jax.ShapeDtypeStruct((NC, N_NODES, L), jnp.float32)),
    mesh=plsc.VectorSubcoreMesh(**_MESH),
    compiler_params=_SC_PARAMS,
    scratch_types=[
        pltpu.VMEM((CHUNK,), jnp.int32),
        pltpu.VMEM((CHUNK,), jnp.int32),
        pltpu.VMEM((CHUNK,), jnp.int32),
        pltpu.VMEM((CHUNK,), jnp.int32),
        pltpu.VMEM((CHUNK, DIM), jnp.float32),
        pltpu.VMEM((CHUNK, DIM), jnp.float32),
        pltpu.VMEM((CHUNK, DIM), jnp.float32),
        pltpu.VMEM((CHUNK, L), jnp.float32),
        pltpu.VMEM((DRAIN, L), jnp.float32),
        pltpu.VMEM_SHARED((N_NODES, DIM), jnp.float32),
        pltpu.VMEM_SHARED((N_NODES, L), jnp.float32),
        pltpu.SemaphoreType.DMA,
        pltpu.SemaphoreType.DMA,
        pltpu.SemaphoreType.DMA,
    ],
)
def _sc_edge_attention(head_hbm, tail_hbm, et_hbm, q_hbm, ktab_hbm, vtab_hbm,
                       kgU_hbm, attN_hbm,
                       headb, tailb, etb, kidxb, qrows, krows, vrows,
                       attb, zbufa, kgU_sh, attN_sh,
                       sem1, sem2, sem3):
    c = lax.axis_index("c")
    s = lax.axis_index("s")
    wid = s * NC + c
    ebase = wid * EPW
    rbase = s * ROWS_PT

    _zero_vmem_2d(vrows, BOUNCE, DIM)
    _zero_vmem_2d(zbufa, DRAIN, L)

    @pl.when(s < NDRAIN_SC)
    def _():
        for j in range(ROWS_PT // BOUNCE):
            pltpu.sync_copy(vrows.at[pl.ds(0, BOUNCE)],
                            kgU_sh.at[pl.ds(rbase + j * BOUNCE, BOUNCE)])
        for j in range(ROWS_PT // DRAIN):
            pltpu.sync_copy(zbufa,
                            attN_sh.at[pl.ds(rbase + j * DRAIN, DRAIN)])

    plsc.subcore_barrier()

    iota = lax.iota(jnp.int32, L)
    perms = [iota ^ st for st in (8, 4, 2, 1)]

    def chunk_body(ci, _):
        off = ebase + ci * CHUNK
        pltpu.sync_copy(head_hbm.at[pl.ds(off, CHUNK)], headb)
        pltpu.sync_copy(tail_hbm.at[pl.ds(off, CHUNK)], tailb)
        pltpu.sync_copy(et_hbm.at[pl.ds(off, CHUNK)], etb)

        def kidx_body(g, _):
            sl = pl.ds(g * L, L)
            kidxb[sl] = etb[sl] * N_NODES + tailb[sl]
            return 0

        lax.fori_loop(0, GROUPS, kidx_body, 0)

        cp1 = pltpu.async_copy(q_hbm.at[headb], qrows, sem1)
        cp2 = pltpu.async_copy(ktab_hbm.at[kidxb], krows, sem2)
        cp3 = pltpu.async_copy(vtab_hbm.at[kidxb], vrows, sem3)
        cp1.wait()
        cp2.wait()
        cp3.wait()

        def edge_body(e, _):
            q = [qrows[e, pl.ds(L * k, L)] for k in range(DIM // L)]
            k = [krows[e, pl.ds(L * k_, L)] for k_ in range(DIM // L)]
            prod = [q[i] * k[i] for i in range(DIM // L)]
            acc = jnp.zeros((L,), jnp.float32)
            eh = []
            for h in range(HEADS):
                x = prod[2 * h] + prod[2 * h + 1]
                for pm in perms:
                    x = x + x[pm]
                ex = jnp.exp(jnp.clip(x, -10.0, 10.0))
                eh.append(ex)
                acc = jnp.where(iota == h, ex, acc)
            attb[e, :] = acc
            for k_ in range(DIM // L):
                sl = pl.ds(L * k_, L)
                vrows[e, sl] = vrows[e, sl] * eh[k_ // 2]
            return 0

        lax.fori_loop(0, CHUNK, edge_body, 0)

        pltpu.sync_copy(vrows, kgU_sh.at[headb], add=True)
        pltpu.sync_copy(attb, attN_sh.at[headb], add=True)
        return 0

    lax.fori_loop(0, NCHUNK, chunk_body, 0)
    plsc.subcore_barrier()

    @pl.when(s < NDRAIN_SC)
    def _():
        for j in range(ROWS_PT // BOUNCE):
            sl = pl.ds(rbase + j * BOUNCE, BOUNCE)
            pltpu.sync_copy(kgU_sh.at[sl], vrows.at[pl.ds(0, BOUNCE)])
            pltpu.sync_copy(vrows.at[pl.ds(0, BOUNCE)], kgU_hbm.at[c, sl])
        for j in range(ROWS_PT // DRAIN):
            sl = pl.ds(rbase + j * DRAIN, DRAIN)
            pltpu.sync_copy(attN_sh.at[sl], zbufa)
            pltpu.sync_copy(zbufa, attN_hbm.at[c, sl])


@functools.partial(
    pl.kernel,
    out_type=(jax.ShapeDtypeStruct((NC, N_NODES, DIM), jnp.float32),
              jax.ShapeDtypeStruct((NC, N_NODES, L), jnp.float32)),
    mesh=plsc.VectorSubcoreMesh(**_MESH),
    compiler_params=_SC_PARAMS,
    scratch_types=[
        pltpu.VMEM((CHUNK,), jnp.int32),
        pltpu.VMEM((CHUNK,), jnp.int32),
        pltpu.VMEM((CHUNK,), jnp.int32),
        pltpu.VMEM((CHUNK, L), jnp.float32),
        pltpu.VMEM((CHUNK, L), jnp.float32),
        pltpu.VMEM((CHUNK, DIM), jnp.float32),
        pltpu.VMEM((CHUNK, L), jnp.float32),
        pltpu.VMEM((DRAIN, L), jnp.float32),
        pltpu.VMEM_SHARED((N_NODES, DIM), jnp.float32),
        pltpu.VMEM_SHARED((N_NODES, L), jnp.float32),
        pltpu.SemaphoreType.DMA,
        pltpu.SemaphoreType.DMA,
        pltpu.SemaphoreType.DMA,
    ],
)
def _sc_edge_weighting(head_hbm, tail_hbm, et_hbm, g_hbm, emb_hbm,
                       outU_hbm, sU_hbm,
                       headb, tailb, etb, ghrows, gtrows, erows, pb,
                       zbufa, outU_sh, sU_sh, sem1, sem2, sem3):
    c = lax.axis_index("c")
    s = lax.axis_index("s")
    wid = s * NC + c
    ebase = wid * EPW
    rbase = s * ROWS_PT

    _zero_vmem_2d(erows, BOUNCE, DIM)
    _zero_vmem_2d(zbufa, DRAIN, L)

    @pl.when(s < NDRAIN_SC)
    def _():
        for j in range(ROWS_PT // BOUNCE):
            pltpu.sync_copy(erows.at[pl.ds(0, BOUNCE)],
                            outU_sh.at[pl.ds(rbase + j * BOUNCE, BOUNCE)])
        for j in range(ROWS_PT // DRAIN):
            pltpu.sync_copy(zbufa, sU_sh.at[pl.ds(rbase + j * DRAIN, DRAIN)])

    plsc.subcore_barrier()

    iota = lax.iota(jnp.int32, L)

    def chunk_body(ci, _):
        off = ebase + ci * CHUNK
        pltpu.sync_copy(head_hbm.at[pl.ds(off, CHUNK)], headb)
        pltpu.sync_copy(tail_hbm.at[pl.ds(off, CHUNK)], tailb)
        pltpu.sync_copy(et_hbm.at[pl.ds(off, CHUNK)], etb)

        cp1 = pltpu.async_copy(g_hbm.at[headb], ghrows, sem1)
        cp2 = pltpu.async_copy(g_hbm.at[tailb], gtrows, sem2)
        cp3 = pltpu.async_copy(emb_hbm.at[tailb], erows, sem3)
        cp1.wait()
        cp2.wait()
        cp3.wait()

        def group_body(g, _):
            rel = etb[pl.ds(g * L, L)]
            for j in range(L):
                e = g * L + j
                ridx = jnp.full((L,), rel[j], jnp.int32)
                a = ghrows[e, :][ridx]
                b = gtrows[e, :][ridx]
                pw = jnp.exp(a * b)
                pb[e, :] = jnp.where(iota == 0, pw, 0.0)
                for k_ in range(DIM // L):
                    sl = pl.ds(L * k_, L)
                    erows[e, sl] = erows[e, sl] * pw
            return 0

        lax.fori_loop(0, GROUPS, group_body, 0)

        pltpu.sync_copy(erows, outU_sh.at[headb], add=True)
        pltpu.sync_copy(pb, sU_sh.at[headb], add=True)
        return 0

    lax.fori_loop(0, NCHUNK, chunk_body, 0)
    plsc.subcore_barrier()

    @pl.when(s < NDRAIN_SC)
    def _():
        for j in range(ROWS_PT // BOUNCE):
            sl = pl.ds(rbase + j * BOUNCE, BOUNCE)
            pltpu.sync_copy(outU_sh.at[sl], erows.at[pl.ds(0, BOUNCE)])
            pltpu.sync_copy(erows.at[pl.ds(0, BOUNCE)], outU_hbm.at[c, sl])
        for j in range(ROWS_PT // DRAIN):
            sl = pl.ds(rbase + j * DRAIN, DRAIN)
            pltpu.sync_copy(sU_sh.at[sl], zbufa)
            pltpu.sync_copy(zbufa, sU_hbm.at[c, sl])


def kernel(entity_emb, weight, qTrans, kTrans, vTrans, edge_index, edge_type):
    head = edge_index[0]
    tail = edge_index[1]
    et = edge_type.astype(jnp.int32)
    wpad = jnp.zeros((L, DIM), jnp.float32).at[:NREL].set(weight)

    q_tab = _tc_q(entity_emb, qTrans)
    k_tab, v_tab = _tc_tables(wpad, entity_emb, kTrans, vTrans)
    kgU, attN = _sc_edge_attention(head, tail, et, q_tab, k_tab, v_tab)
    g_tab = _tc_finalize_g(wpad, kgU, attN)
    outU, sU = _sc_edge_weighting(head, tail, et, g_tab, entity_emb)
    return _tc_final(outU, sU)
